# baseline (speedup 1.0000x reference)
import functools

import jax
import jax.numpy as jnp
from jax import lax
from jax.experimental import pallas as pl
from jax.experimental.pallas import tpu as pltpu
from jax.experimental.pallas import tpu_sc as plsc

_B = 8
_L = 2048
_LANES = 16
_NC = 2
_NS = 16
_NW = _NC * _NS
_K = 8
_MSE_W = 1.0


def _tec_body(pred_hbm, true_hbm, mask_hbm, len_hbm, out_hbm,
              mask_v, lbuf, pbuf, tbuf, obuf):
    wid = lax.axis_index("s") * _NC + lax.axis_index("c")

    pltpu.sync_copy(mask_hbm, mask_v)
    pltpu.sync_copy(len_hbm, lbuf)

    lvec = lbuf[...]
    total = jnp.zeros((_LANES,), jnp.float32)
    for b in range(_B):
        len_i = lvec[b]
        len_f = len_i.astype(jnp.float32)
        den = jnp.broadcast_to(len_f * len_f, (_LANES,))
        w_b = 1.0 / (_B * jnp.maximum(den, 1.0))

        nchunks = (len_i + (_K - 1)) // _K
        num_my = jnp.maximum(nchunks - wid + (_NW - 1), 0) // _NW
        jceil = (len_i + (_LANES - 1)) // _LANES

        def _chunk_body(t, acc):
            cidx = wid + t * _NW
            r0 = cidx * _K
            pltpu.sync_copy(pred_hbm.at[b, pl.ds(r0, _K), :], pbuf)
            pltpu.sync_copy(true_hbm.at[b, pl.ds(r0, _K), :], tbuf)
            nrows = jnp.minimum(_K, len_i - r0)

            def _row_body(rr, acc_r):
                def _col_body(j, acc_c):
                    c0 = j * _LANES
                    m = mask_v[b, pl.ds(c0, _LANES)]
                    d = pbuf[rr, pl.ds(c0, _LANES)] - tbuf[rr, pl.ds(c0, _LANES)]
                    return acc_c + m * d * d
                return lax.fori_loop(0, jceil, _col_body, acc_r)

            return lax.fori_loop(0, nrows, _row_body, acc)

        s_acc = lax.fori_loop(0, num_my, _chunk_body,
                              jnp.zeros((_LANES,), jnp.float32))
        total = total + s_acc * w_b

    obuf[0, :] = total
    pltpu.sync_copy(obuf, out_hbm.at[pl.ds(wid, 1)])


@jax.jit
def _sc_partials(pred, true, mask, lengths):
    mesh = plsc.VectorSubcoreMesh(core_axis_name="c", subcore_axis_name="s")
    return pl.kernel(
        _tec_body,
        out_type=jax.ShapeDtypeStruct((_NW, _LANES), jnp.float32),
        mesh=mesh,
        scratch_types=[
            pltpu.VMEM((_B, _L), jnp.float32),
            pltpu.VMEM((_LANES,), jnp.int32),
            pltpu.VMEM((_K, _L), jnp.float32),
            pltpu.VMEM((_K, _L), jnp.float32),
            pltpu.VMEM((1, _LANES), jnp.float32),
        ],
    )(pred, true, mask, lengths)


def kernel(pred_corrs, true_corrs, resi_pad_mask):
    lengths = jnp.zeros((_LANES,), jnp.int32).at[:_B].set(
        jnp.sum(resi_pad_mask, axis=1).astype(jnp.int32))
    partials = _sc_partials(pred_corrs, true_corrs, resi_pad_mask, lengths)
    mse = jnp.sum(partials)
    return (_MSE_W * mse, mse)


# speedup vs baseline: 1.5802x; 1.5802x over previous
import functools

import jax
import jax.numpy as jnp
from jax import lax
from jax.experimental import pallas as pl
from jax.experimental.pallas import tpu as pltpu
from jax.experimental.pallas import tpu_sc as plsc

_B = 8
_L = 2048
_LANES = 16
_NC = 2
_NS = 16
_NW = _NC * _NS
_K = 16
_CW = 256
_NG = _CW // _LANES
_MSE_W = 1.0


def _tec_body(pred_hbm, true_hbm, mask_hbm, len_hbm, out_hbm,
              mask_v, lbuf, pbuf, tbuf, obuf, uacc, psem, tsem):
    wid = lax.axis_index("s") * _NC + lax.axis_index("c")

    pltpu.sync_copy(mask_hbm, mask_v)
    pltpu.sync_copy(len_hbm, lbuf)
    lvec = lbuf[...]

    total = jnp.zeros((_LANES,), jnp.float32)
    for b in range(_B):
        len_i = lvec[b]
        len_f = len_i.astype(jnp.float32)
        den = jnp.broadcast_to(len_f * len_f, (_LANES,))
        w_b = 1.0 / (_B * jnp.maximum(den, 1.0))

        nchunks = (len_i + (_K - 1)) // _K
        ncb = (len_i + (_CW - 1)) // _CW
        num_my = jnp.maximum(nchunks - wid + (_NW - 1), 0) // _NW
        nunits = num_my * ncb

        def _unit_rc(u):
            t = u // ncb
            cb = u % ncb
            r0 = (wid + t * _NW) * _K
            return r0, cb * _CW

        def _start(u, slot):
            r0, c0 = _unit_rc(u)
            pltpu.async_copy(pred_hbm.at[b, pl.ds(r0, _K), pl.ds(c0, _CW)],
                             pbuf.at[slot], psem.at[slot])
            pltpu.async_copy(true_hbm.at[b, pl.ds(r0, _K), pl.ds(c0, _CW)],
                             tbuf.at[slot], tsem.at[slot])

        def _wait(u, slot):
            r0, c0 = _unit_rc(u)
            pltpu.make_async_copy(
                pred_hbm.at[b, pl.ds(r0, _K), pl.ds(c0, _CW)],
                pbuf.at[slot], psem.at[slot]).wait()
            pltpu.make_async_copy(
                true_hbm.at[b, pl.ds(r0, _K), pl.ds(c0, _CW)],
                tbuf.at[slot], tsem.at[slot]).wait()

        @pl.when(nunits > 0)
        def _():
            _start(0, 0)

        def _unit_body(u, acc):
            slot = u % 2
            _wait(u, slot)

            @pl.when(u + 1 < nunits)
            def _():
                _start(u + 1, (u + 1) % 2)

            r0, c0 = _unit_rc(u)
            nrows = jnp.minimum(_K, len_i - r0)
            interior = c0 + _CW <= len_i

            def _row_full(rr, acc_r):
                a = acc_r
                for g in range(_NG):
                    d = (pbuf[slot, rr, pl.ds(g * _LANES, _LANES)]
                         - tbuf[slot, rr, pl.ds(g * _LANES, _LANES)])
                    a = a + d * d
                return a

            def _row_masked(rr, acc_r):
                a = acc_r
                for g in range(_NG):
                    m = mask_v[b, pl.ds(c0 + g * _LANES, _LANES)]
                    d = (pbuf[slot, rr, pl.ds(g * _LANES, _LANES)]
                         - tbuf[slot, rr, pl.ds(g * _LANES, _LANES)])
                    a = a + m * (d * d)
                return a

            @pl.when(interior)
            def _():
                uacc[...] = lax.fori_loop(
                    0, nrows, _row_full, jnp.zeros((_LANES,), jnp.float32))

            @pl.when(jnp.logical_not(interior))
            def _():
                uacc[...] = lax.fori_loop(
                    0, nrows, _row_masked, jnp.zeros((_LANES,), jnp.float32))

            return acc + uacc[...]

        s_acc = lax.fori_loop(0, nunits, _unit_body,
                              jnp.zeros((_LANES,), jnp.float32))
        total = total + s_acc * w_b

    obuf[0, :] = total
    pltpu.sync_copy(obuf, out_hbm.at[pl.ds(wid, 1)])


@jax.jit
def _sc_partials(pred, true, mask, lengths):
    mesh = plsc.VectorSubcoreMesh(core_axis_name="c", subcore_axis_name="s")
    return pl.kernel(
        _tec_body,
        out_type=jax.ShapeDtypeStruct((_NW, _LANES), jnp.float32),
        mesh=mesh,
        scratch_types=[
            pltpu.VMEM((_B, _L), jnp.float32),
            pltpu.VMEM((_LANES,), jnp.int32),
            pltpu.VMEM((2, _K, _CW), jnp.float32),
            pltpu.VMEM((2, _K, _CW), jnp.float32),
            pltpu.VMEM((1, _LANES), jnp.float32),
            pltpu.VMEM((_LANES,), jnp.float32),
            pltpu.SemaphoreType.DMA((2,)),
            pltpu.SemaphoreType.DMA((2,)),
        ],
    )(pred, true, mask, lengths)


def kernel(pred_corrs, true_corrs, resi_pad_mask):
    lengths = jnp.zeros((_LANES,), jnp.int32).at[:_B].set(
        jnp.sum(resi_pad_mask, axis=1).astype(jnp.int32))
    partials = _sc_partials(pred_corrs, true_corrs, resi_pad_mask, lengths)
    mse = jnp.sum(partials)
    return (_MSE_W * mse, mse)


# speedup vs baseline: 1.9422x; 1.2291x over previous
import functools

import jax
import jax.numpy as jnp
from jax import lax
from jax.experimental import pallas as pl
from jax.experimental.pallas import tpu as pltpu
from jax.experimental.pallas import tpu_sc as plsc

_B = 8
_L = 2048
_LANES = 16
_NC = 2
_NS = 16
_NW = _NC * _NS
_K = 16
_CW = 512
_NG = _CW // _LANES
_MSE_W = 1.0


def _tec_body(pred_hbm, true_hbm, mask_hbm, len_hbm, out_hbm,
              mask_v, lbuf, pbuf, tbuf, obuf, uacc, psem, tsem):
    wid = lax.axis_index("s") * _NC + lax.axis_index("c")

    pltpu.sync_copy(mask_hbm, mask_v)
    pltpu.sync_copy(len_hbm, lbuf)
    lvec = lbuf[...]

    total = jnp.zeros((_LANES,), jnp.float32)
    for b in range(_B):
        len_i = lvec[b]
        len_f = len_i.astype(jnp.float32)
        den = jnp.broadcast_to(len_f * len_f, (_LANES,))
        w_b = 1.0 / (_B * jnp.maximum(den, 1.0))

        nchunks = (len_i + (_K - 1)) // _K
        ncb = (len_i + (_CW - 1)) // _CW
        num_my = jnp.maximum(nchunks - wid + (_NW - 1), 0) // _NW
        nunits = num_my * ncb

        def _unit_rc(u):
            t = u // ncb
            cb = u % ncb
            r0 = (wid + t * _NW) * _K
            return r0, cb * _CW

        def _start(u, slot):
            r0, c0 = _unit_rc(u)
            pltpu.async_copy(pred_hbm.at[b, pl.ds(r0, _K), pl.ds(c0, _CW)],
                             pbuf.at[slot], psem.at[slot])
            pltpu.async_copy(true_hbm.at[b, pl.ds(r0, _K), pl.ds(c0, _CW)],
                             tbuf.at[slot], tsem.at[slot])

        def _wait(u, slot):
            r0, c0 = _unit_rc(u)
            pltpu.make_async_copy(
                pred_hbm.at[b, pl.ds(r0, _K), pl.ds(c0, _CW)],
                pbuf.at[slot], psem.at[slot]).wait()
            pltpu.make_async_copy(
                true_hbm.at[b, pl.ds(r0, _K), pl.ds(c0, _CW)],
                tbuf.at[slot], tsem.at[slot]).wait()

        @pl.when(nunits > 0)
        def _():
            _start(0, 0)

        def _unit_body(u, acc):
            slot = u % 2
            _wait(u, slot)

            @pl.when(u + 1 < nunits)
            def _():
                _start(u + 1, (u + 1) % 2)

            r0, c0 = _unit_rc(u)
            nrows = jnp.minimum(_K, len_i - r0)
            interior = c0 + _CW <= len_i

            def _row_full(rr, acc_r):
                a = acc_r
                for g in range(_NG):
                    d = (pbuf[slot, rr, pl.ds(g * _LANES, _LANES)]
                         - tbuf[slot, rr, pl.ds(g * _LANES, _LANES)])
                    a = a + d * d
                return a

            def _row_masked(rr, acc_r):
                a = acc_r
                for g in range(_NG):
                    m = mask_v[b, pl.ds(c0 + g * _LANES, _LANES)]
                    d = (pbuf[slot, rr, pl.ds(g * _LANES, _LANES)]
                         - tbuf[slot, rr, pl.ds(g * _LANES, _LANES)])
                    a = a + m * (d * d)
                return a

            @pl.when(interior)
            def _():
                uacc[...] = lax.fori_loop(
                    0, nrows, _row_full, jnp.zeros((_LANES,), jnp.float32))

            @pl.when(jnp.logical_not(interior))
            def _():
                uacc[...] = lax.fori_loop(
                    0, nrows, _row_masked, jnp.zeros((_LANES,), jnp.float32))

            return acc + uacc[...]

        s_acc = lax.fori_loop(0, nunits, _unit_body,
                              jnp.zeros((_LANES,), jnp.float32))
        total = total + s_acc * w_b

    obuf[0, :] = total
    pltpu.sync_copy(obuf, out_hbm.at[pl.ds(wid, 1)])


@jax.jit
def _sc_partials(pred, true, mask, lengths):
    mesh = plsc.VectorSubcoreMesh(core_axis_name="c", subcore_axis_name="s")
    return pl.kernel(
        _tec_body,
        out_type=jax.ShapeDtypeStruct((_NW, _LANES), jnp.float32),
        mesh=mesh,
        scratch_types=[
            pltpu.VMEM((_B, _L), jnp.float32),
            pltpu.VMEM((_LANES,), jnp.int32),
            pltpu.VMEM((2, _K, _CW), jnp.float32),
            pltpu.VMEM((2, _K, _CW), jnp.float32),
            pltpu.VMEM((1, _LANES), jnp.float32),
            pltpu.VMEM((_LANES,), jnp.float32),
            pltpu.SemaphoreType.DMA((2,)),
            pltpu.SemaphoreType.DMA((2,)),
        ],
    )(pred, true, mask, lengths)


def kernel(pred_corrs, true_corrs, resi_pad_mask):
    lengths = jnp.zeros((_LANES,), jnp.int32).at[:_B].set(
        jnp.sum(resi_pad_mask, axis=1).astype(jnp.int32))
    partials = _sc_partials(pred_corrs, true_corrs, resi_pad_mask, lengths)
    mse = jnp.sum(partials)
    return (_MSE_W * mse, mse)
